# baseline (speedup 1.0000x reference)
import functools

import jax
import jax.numpy as jnp
from jax import lax
from jax.experimental import pallas as pl
from jax.experimental.pallas import tpu as pltpu
from jax.experimental.pallas import tpu_sc as plsc

_NW = 32
_CHUNK = 2048


def _emb_call(x1, w1, b, s, d, vp):
    n = x1.shape[0]
    nchunks = n // _CHUNK
    d_per = d // _NW
    per_s = d * b

    mesh = plsc.VectorSubcoreMesh(core_axis_name="c", subcore_axis_name="s")

    @functools.partial(
        pl.kernel,
        mesh=mesh,
        out_type=jax.ShapeDtypeStruct((s * d * b,), jnp.float32),
        compiler_params=pltpu.CompilerParams(
            use_tc_tiling_on_sc=False, needs_layout_passes=False
        ),
        scratch_types=[
            pltpu.VMEM((vp,), jnp.float32),
            pltpu.VMEM((2, _CHUNK), jnp.int32),
            pltpu.VMEM((2, _CHUNK), jnp.float32),
            pltpu.SemaphoreType.DMA((2,)),
            pltpu.SemaphoreType.DMA((2,)),
        ],
    )
    def emb(x_hbm, w_hbm, out_hbm, wrow_v, idx_v, out_v, isem, osem):
        wid = lax.axis_index("s") * 2 + lax.axis_index("c")

        def start_idx(c, pp):
            pltpu.async_copy(
                x_hbm.at[pl.ds(c * _CHUNK, _CHUNK)], idx_v.at[pp], isem.at[pp]
            )

        def wait_idx(pp):
            pltpu.make_async_copy(
                x_hbm.at[pl.ds(0, _CHUNK)], idx_v.at[pp], isem.at[pp]
            ).wait()

        def out_base(c, dd):
            return (c // 2) * per_s + dd * b + (c % 2) * _CHUNK

        def start_out(c, dd, pp):
            pltpu.async_copy(
                out_v.at[pp], out_hbm.at[pl.ds(out_base(c, dd), _CHUNK)], osem.at[pp]
            )

        def wait_out(pp):
            pltpu.make_async_copy(
                out_v.at[pp], out_hbm.at[pl.ds(0, _CHUNK)], osem.at[pp]
            ).wait()

        def gather_chunk(pp):
            for k in range(_CHUNK // 16):
                iv = idx_v[pp, pl.ds(k * 16, 16)]
                out_v[pp, pl.ds(k * 16, 16)] = plsc.load_gather(wrow_v, [iv])

        for dloc in range(d_per):
            dd = wid * d_per + dloc
            pltpu.sync_copy(w_hbm.at[pl.ds(dd * vp, vp)], wrow_v)
            start_idx(0, 0)
            start_idx(1, 1)

            def body(cc, carry):
                for pp in range(2):
                    c = cc * 2 + pp
                    wait_idx(pp)

                    @pl.when(cc > 0)
                    def _():
                        wait_out(pp)

                    gather_chunk(pp)

                    @pl.when(cc < nchunks // 2 - 1)
                    def _():
                        start_idx(c + 2, pp)

                    start_out(c, dd, pp)
                return carry

            lax.fori_loop(0, nchunks // 2, body, 0)
            wait_out(0)
            wait_out(1)

    return emb(x1, w1)


def kernel(x, weight):
    b, s = x.shape
    v, d = weight.shape
    x1 = x.T.astype(jnp.int32).reshape(-1)
    w1 = weight.T.reshape(-1)
    out1 = _emb_call(x1, w1, b, s, d, v)
    return out1.reshape(s, d, b).transpose(2, 0, 1)


# speedup vs baseline: 1.0745x; 1.0745x over previous
import functools

import jax
import jax.numpy as jnp
from jax import lax
from jax.experimental import pallas as pl
from jax.experimental.pallas import tpu as pltpu
from jax.experimental.pallas import tpu_sc as plsc

_GROUP = 16
_NBUF = 2
_NW = 32


def _emb_call(x, weight):
    b, s = x.shape
    _, d = weight.shape
    rows_per_tile = b // _NW
    groups = rows_per_tile // _GROUP

    mesh = plsc.VectorSubcoreMesh(core_axis_name="c", subcore_axis_name="s")

    @functools.partial(
        pl.kernel,
        mesh=mesh,
        out_type=jax.ShapeDtypeStruct((b, s, d), jnp.float32),
        compiler_params=pltpu.CompilerParams(use_tc_tiling_on_sc=False),
        scratch_types=[
            pltpu.VMEM((rows_per_tile, s), jnp.int32),
            pltpu.VMEM((_NBUF, _GROUP, s, d), jnp.float32),
            pltpu.SemaphoreType.DMA((_NBUF,)),
            pltpu.SemaphoreType.DMA((_NBUF,)),
        ],
    )
    def emb(x_hbm, w_hbm, out_hbm, idx_v, rows_v, gsem, ssem):
        wid = lax.axis_index("s") * 2 + lax.axis_index("c")
        row0 = wid * rows_per_tile
        pltpu.sync_copy(x_hbm.at[pl.ds(row0, rows_per_tile)], idx_v)

        def start_gathers(g, p):
            for q in range(_GROUP):
                pltpu.async_copy(
                    w_hbm.at[idx_v.at[g * _GROUP + q]],
                    rows_v.at[p].at[q],
                    gsem.at[p],
                )

        def out_slice(g):
            return out_hbm.at[pl.ds(row0 + g * _GROUP, _GROUP)]

        def wait_gathers(g, p):
            pltpu.make_async_copy(out_slice(g), rows_v.at[p], gsem.at[p]).wait()

        def start_store(g, p):
            pltpu.async_copy(rows_v.at[p], out_slice(g), ssem.at[p])

        def wait_store(g, p):
            pltpu.make_async_copy(rows_v.at[p], out_slice(g), ssem.at[p]).wait()

        for p in range(_NBUF):
            start_gathers(p, p)

        def body(pp, carry):
            for p in range(_NBUF):
                g = pp * _NBUF + p
                wait_gathers(g, p)
                start_store(g, p)
                wait_store(g, p)
                start_gathers(g + _NBUF, p)
            return carry

        lax.fori_loop(0, groups // _NBUF - 1, body, 0)

        for p in range(_NBUF):
            g = groups - _NBUF + p
            wait_gathers(g, p)
            start_store(g, p)
        for p in range(_NBUF):
            wait_store(groups - _NBUF + p, p)

    return emb(x, weight)


def kernel(x, weight):
    b, s = x.shape
    v, d = weight.shape
    x2 = x.astype(jnp.int32).reshape(-1).reshape(b, s)
    w2 = weight.reshape(-1).reshape(v, d)
    return _emb_call(x2, w2)


# speedup vs baseline: 1.4684x; 1.3666x over previous
import functools

import jax
import jax.numpy as jnp
from jax import lax
from jax.experimental import pallas as pl
from jax.experimental.pallas import tpu as pltpu
from jax.experimental.pallas import tpu_sc as plsc

_NW = 32
_CHUNK = 2048


def _emb_call(x1, w1, b, s, d, v):
    n = x1.shape[0]
    nchunks = n // _CHUNK
    d_per = d // _NW
    nb = b // 128
    runs = _CHUNK // 128

    mesh = plsc.VectorSubcoreMesh(core_axis_name="c", subcore_axis_name="s")

    @functools.partial(
        pl.kernel,
        mesh=mesh,
        out_type=jax.ShapeDtypeStruct((s, d // 8, nb, 8 * 128), jnp.float32),
        compiler_params=pltpu.CompilerParams(
            use_tc_tiling_on_sc=False, needs_layout_passes=False
        ),
        scratch_types=[
            pltpu.VMEM((v,), jnp.float32),
            pltpu.VMEM((2, _CHUNK), jnp.int32),
            pltpu.VMEM((2, runs, 128), jnp.float32),
            pltpu.SemaphoreType.DMA((2,)),
            pltpu.SemaphoreType.DMA((2,)),
        ],
    )
    def emb(x_hbm, w_hbm, out_hbm, wrow_v, idx_v, out_v, isem, osem):
        wid = lax.axis_index("s") * 2 + lax.axis_index("c")

        def start_idx(c, pp):
            pltpu.async_copy(
                x_hbm.at[pl.ds(c * _CHUNK, _CHUNK)], idx_v.at[pp], isem.at[pp]
            )

        def wait_idx(pp):
            pltpu.make_async_copy(
                x_hbm.at[pl.ds(0, _CHUNK)], idx_v.at[pp], isem.at[pp]
            ).wait()

        def out_slice(c, dd):
            return out_hbm.at[
                c // 2,
                dd // 8,
                pl.ds((c % 2) * runs, runs),
                pl.ds(pl.multiple_of((dd % 8) * 128, 128), 128),
            ]

        def start_out(c, dd, pp):
            pltpu.async_copy(out_v.at[pp], out_slice(c, dd), osem.at[pp])

        def wait_out(c, dd, pp):
            pltpu.make_async_copy(out_v.at[pp], out_slice(c, dd), osem.at[pp]).wait()

        def gather_chunk(pp):
            @plsc.parallel_loop(0, runs)
            def _(r):
                for kk in range(8):
                    iv = idx_v[pp, pl.ds(r * 128 + kk * 16, 16)]
                    out_v[pp, r, pl.ds(kk * 16, 16)] = plsc.load_gather(
                        wrow_v, [iv]
                    )

        for dloc in range(d_per):
            dd = wid * d_per + dloc
            pltpu.sync_copy(w_hbm.at[pl.ds(dd * v, v)], wrow_v)
            start_idx(0, 0)
            start_idx(1, 1)

            def body(cc, carry):
                for pp in range(2):
                    c = cc * 2 + pp

                    wait_idx(pp)

                    @pl.when(cc > 0)
                    def _():
                        wait_out(c - 2, dd, pp)

                    gather_chunk(pp)

                    @pl.when(cc < nchunks // 2 - 1)
                    def _():
                        start_idx(c + 2, pp)

                    start_out(c, dd, pp)
                return carry

            lax.fori_loop(0, nchunks // 2, body, 0)
            wait_out(nchunks - 2, dd, 0)
            wait_out(nchunks - 1, dd, 1)

    return emb(x1, w1)


def kernel(x, weight):
    b, s = x.shape
    v, d = weight.shape
    x1 = x.T.astype(jnp.int32).reshape(-1)
    w1 = weight.T.reshape(-1)
    out4 = _emb_call(x1, w1, b, s, d, v)
    out5 = out4.reshape(s, d // 8, b // 128, 8, 128)
    return out5.transpose(2, 4, 0, 1, 3).reshape(b, s, d)


# speedup vs baseline: 1.4857x; 1.0118x over previous
import functools

import jax
import jax.numpy as jnp
from jax import lax
from jax.experimental import pallas as pl
from jax.experimental.pallas import tpu as pltpu
from jax.experimental.pallas import tpu_sc as plsc

_NW = 32
_CHUNK = 2048


def _emb_call(x1, w1, b, s, d, v):
    n = x1.shape[0]
    nchunks = n // _CHUNK
    d_per = d // _NW
    nb = b // 128
    runs = _CHUNK // 128

    mesh = plsc.VectorSubcoreMesh(core_axis_name="c", subcore_axis_name="s")

    @functools.partial(
        pl.kernel,
        mesh=mesh,
        out_type=jax.ShapeDtypeStruct((s, d // 8, nb, 8 * 128), jnp.float32),
        compiler_params=pltpu.CompilerParams(
            use_tc_tiling_on_sc=False, needs_layout_passes=False
        ),
        scratch_types=[
            pltpu.VMEM((v,), jnp.float32),
            pltpu.VMEM((2, _CHUNK), jnp.int32),
            pltpu.VMEM((2, runs, 128), jnp.float32),
            pltpu.SemaphoreType.DMA((2,)),
            pltpu.SemaphoreType.DMA((2,)),
        ],
    )
    def emb(x_hbm, w_hbm, out_hbm, wrow_v, idx_v, out_v, isem, osem):
        wid = lax.axis_index("s") * 2 + lax.axis_index("c")

        def start_idx(c, pp):
            pltpu.async_copy(
                x_hbm.at[pl.ds(c * _CHUNK, _CHUNK)], idx_v.at[pp], isem.at[pp]
            )

        def wait_idx(pp):
            pltpu.make_async_copy(
                x_hbm.at[pl.ds(0, _CHUNK)], idx_v.at[pp], isem.at[pp]
            ).wait()

        def out_slice(c, dd):
            return out_hbm.at[
                c // 2,
                dd // 8,
                pl.ds((c % 2) * runs, runs),
                pl.ds(pl.multiple_of((dd % 8) * 128, 128), 128),
            ]

        def start_out(c, dd, pp):
            pltpu.async_copy(out_v.at[pp], out_slice(c, dd), osem.at[pp])

        def wait_out(c, dd, pp):
            pltpu.make_async_copy(out_v.at[pp], out_slice(c, dd), osem.at[pp]).wait()

        def gather_chunk(pp):
            @plsc.parallel_loop(0, _CHUNK, 16, unroll=8)
            def _(i):
                iv = idx_v[pp, pl.ds(i, 16)]
                out_v[pp, i // 128, pl.ds(i % 128, 16)] = plsc.load_gather(
                    wrow_v, [iv]
                )

        for dloc in range(d_per):
            dd = wid * d_per + dloc
            pltpu.sync_copy(w_hbm.at[pl.ds(dd * v, v)], wrow_v)
            start_idx(0, 0)
            start_idx(1, 1)

            def body(cc, carry):
                for pp in range(2):
                    c = cc * 2 + pp

                    wait_idx(pp)

                    @pl.when(cc > 0)
                    def _():
                        wait_out(c - 2, dd, pp)

                    gather_chunk(pp)

                    @pl.when(cc < nchunks // 2 - 1)
                    def _():
                        start_idx(c + 2, pp)

                    start_out(c, dd, pp)
                return carry

            lax.fori_loop(0, nchunks // 2, body, 0)
            wait_out(nchunks - 2, dd, 0)
            wait_out(nchunks - 1, dd, 1)

    return emb(x1, w1)


def kernel(x, weight):
    b, s = x.shape
    v, d = weight.shape
    x1 = x.T.astype(jnp.int32).reshape(-1)
    w1 = weight.T.reshape(-1)
    out4 = _emb_call(x1, w1, b, s, d, v)
    out5 = out4.reshape(s, d // 8, b // 128, 8, 128)
    return out5.transpose(2, 4, 0, 1, 3).reshape(b, s, d)


# speedup vs baseline: 1.7326x; 1.1662x over previous
import functools

import jax
import jax.numpy as jnp
from jax import lax
from jax.experimental import pallas as pl
from jax.experimental.pallas import tpu as pltpu
from jax.experimental.pallas import tpu_sc as plsc

_NW = 32
_CHUNK = 4096


def _emb_call(x1, w1, b, s, d, v):
    n = x1.shape[0]
    nchunks = n // _CHUNK
    d_per = d // _NW
    nb = b // 128
    runs = _CHUNK // 128

    mesh = plsc.VectorSubcoreMesh(core_axis_name="c", subcore_axis_name="s")

    @functools.partial(
        pl.kernel,
        mesh=mesh,
        out_type=jax.ShapeDtypeStruct((s, d // 8, nb, 8 * 128), jnp.float32),
        compiler_params=pltpu.CompilerParams(
            use_tc_tiling_on_sc=False, needs_layout_passes=False
        ),
        scratch_types=[
            pltpu.VMEM((v,), jnp.float32),
            pltpu.VMEM((2, _CHUNK), jnp.int32),
            pltpu.VMEM((2, runs, 128), jnp.float32),
            pltpu.SemaphoreType.DMA((2,)),
            pltpu.SemaphoreType.DMA((2,)),
        ],
    )
    def emb(x_hbm, w_hbm, out_hbm, wrow_v, idx_v, out_v, isem, osem):
        wid = lax.axis_index("s") * 2 + lax.axis_index("c")

        def start_idx(c, pp):
            pltpu.async_copy(
                x_hbm.at[pl.ds(c * _CHUNK, _CHUNK)], idx_v.at[pp], isem.at[pp]
            )

        def wait_idx(pp):
            pltpu.make_async_copy(
                x_hbm.at[pl.ds(0, _CHUNK)], idx_v.at[pp], isem.at[pp]
            ).wait()

        cps = b // _CHUNK

        def out_slice(c, dd):
            return out_hbm.at[
                c // cps,
                dd // 8,
                pl.ds((c % cps) * runs, runs),
                pl.ds(pl.multiple_of((dd % 8) * 128, 128), 128),
            ]

        def start_out(c, dd, pp):
            pltpu.async_copy(out_v.at[pp], out_slice(c, dd), osem.at[pp])

        def wait_out(c, dd, pp):
            pltpu.make_async_copy(out_v.at[pp], out_slice(c, dd), osem.at[pp]).wait()

        def gather_chunk(pp):
            @plsc.parallel_loop(0, _CHUNK, 16, unroll=8)
            def _(i):
                iv = idx_v[pp, pl.ds(i, 16)]
                out_v[pp, i // 128, pl.ds(i % 128, 16)] = plsc.load_gather(
                    wrow_v, [iv]
                )

        for dloc in range(d_per):
            dd = wid * d_per + dloc
            pltpu.sync_copy(w_hbm.at[pl.ds(dd * v, v)], wrow_v)
            start_idx(0, 0)
            start_idx(1, 1)

            def body(cc, carry):
                for pp in range(2):
                    c = cc * 2 + pp

                    wait_idx(pp)

                    @pl.when(cc > 0)
                    def _():
                        wait_out(c - 2, dd, pp)

                    gather_chunk(pp)

                    @pl.when(cc < nchunks // 2 - 1)
                    def _():
                        start_idx(c + 2, pp)

                    start_out(c, dd, pp)
                return carry

            lax.fori_loop(0, nchunks // 2, body, 0)
            wait_out(nchunks - 2, dd, 0)
            wait_out(nchunks - 1, dd, 1)

    return emb(x1, w1)


def kernel(x, weight):
    b, s = x.shape
    v, d = weight.shape
    x1 = x.T.astype(jnp.int32).reshape(-1)
    w1 = weight.T.reshape(-1)
    out4 = _emb_call(x1, w1, b, s, d, v)
    out5 = out4.reshape(s, d // 8, b // 128, 8, 128)
    return out5.transpose(2, 4, 0, 1, 3).reshape(b, s, d)


# speedup vs baseline: 1.7375x; 1.0028x over previous
import functools

import jax
import jax.numpy as jnp
from jax import lax
from jax.experimental import pallas as pl
from jax.experimental.pallas import tpu as pltpu
from jax.experimental.pallas import tpu_sc as plsc

_NW = 32
_CHUNK = 4096


def _emb_call(x1, w1, b, s, d, v):
    n = x1.shape[0]
    nchunks = n // _CHUNK
    d_per = d // _NW
    nb = b // 128
    runs = _CHUNK // 128

    mesh = plsc.VectorSubcoreMesh(core_axis_name="c", subcore_axis_name="s")

    @functools.partial(
        pl.kernel,
        mesh=mesh,
        out_type=jax.ShapeDtypeStruct((s, d // 8, nb, 8 * 128), jnp.float32),
        compiler_params=pltpu.CompilerParams(
            use_tc_tiling_on_sc=False, needs_layout_passes=False
        ),
        scratch_types=[
            pltpu.VMEM((v,), jnp.float32),
            pltpu.VMEM((2, _CHUNK), jnp.int32),
            pltpu.VMEM((2, runs, 128), jnp.float32),
            pltpu.SemaphoreType.DMA((2,)),
            pltpu.SemaphoreType.DMA((2,)),
        ],
    )
    def emb(x_hbm, w_hbm, out_hbm, wrow_v, idx_v, out_v, isem, osem):
        wid = lax.axis_index("s") * 2 + lax.axis_index("c")

        def start_idx(c, pp):
            pltpu.async_copy(
                x_hbm.at[pl.ds(c * _CHUNK, _CHUNK)], idx_v.at[pp], isem.at[pp]
            )

        def wait_idx(pp):
            pltpu.make_async_copy(
                x_hbm.at[pl.ds(0, _CHUNK)], idx_v.at[pp], isem.at[pp]
            ).wait()

        cps = b // _CHUNK

        def out_slice(c, dd):
            return out_hbm.at[
                c // cps,
                dd // 8,
                pl.ds((c % cps) * runs, runs),
                pl.ds(pl.multiple_of((dd % 8) * 128, 128), 128),
            ]

        def start_out(c, dd, pp):
            pltpu.async_copy(out_v.at[pp], out_slice(c, dd), osem.at[pp])

        def wait_out(c, dd, pp):
            pltpu.make_async_copy(out_v.at[pp], out_slice(c, dd), osem.at[pp]).wait()

        def gather_chunk(pp):
            @plsc.parallel_loop(0, _CHUNK, 16, unroll=16)
            def _(i):
                iv = idx_v[pp, pl.ds(i, 16)]
                out_v[pp, i // 128, pl.ds(i % 128, 16)] = plsc.load_gather(
                    wrow_v, [iv]
                )

        for dloc in range(d_per):
            dd = wid * d_per + dloc
            pltpu.sync_copy(w_hbm.at[pl.ds(dd * v, v)], wrow_v)
            start_idx(0, 0)
            start_idx(1, 1)

            def body(cc, carry):
                for pp in range(2):
                    c = cc * 2 + pp

                    wait_idx(pp)

                    @pl.when(cc > 0)
                    def _():
                        wait_out(c - 2, dd, pp)

                    gather_chunk(pp)

                    @pl.when(cc < nchunks // 2 - 1)
                    def _():
                        start_idx(c + 2, pp)

                    start_out(c, dd, pp)
                return carry

            lax.fori_loop(0, nchunks // 2, body, 0)
            wait_out(nchunks - 2, dd, 0)
            wait_out(nchunks - 1, dd, 1)

    return emb(x1, w1)


def kernel(x, weight):
    b, s = x.shape
    v, d = weight.shape
    x1 = x.T.astype(jnp.int32).reshape(-1)
    w1 = weight.T.reshape(-1)
    out4 = _emb_call(x1, w1, b, s, d, v)
    out5 = out4.reshape(s, d // 8, b // 128, 8, 128)
    return out5.transpose(2, 4, 0, 1, 3).reshape(b, s, d)
